# baseline (speedup 1.0000x reference)
import functools
import jax
import jax.numpy as jnp
from jax import lax
from jax.experimental import pallas as pl
from jax.experimental.pallas import tpu as pltpu
from jax.experimental.pallas import tpu_sc as plsc

B = 16384
INPUT_DIM = 128
EMBED_DIM = 16
BLK = 8192
GRID = B // BLK

_info = plsc.get_sparse_core_info()
_NC, _NS = _info.num_cores, _info.num_subcores
_NW = _NC * _NS
_BPW = B // _NW
_ICH = 128
_NICH = _BPW // _ICH


def _p_kernel(emb_ref, w2_ref, b_ref, p_ref):
    p_ref[...] = jax.lax.dot_general(
        emb_ref[...], w2_ref[...],
        dimension_numbers=(((1,), (1,)), ((), ())),
        preferred_element_type=jnp.float32,
    ) + b_ref[...]


def _sc_gather_kernel(p_hbm, floors_hbm, out_hbm, idx_v, rows_v, sem):
    wid = lax.axis_index("s") * _NC + lax.axis_index("c")
    base = wid * _BPW
    pltpu.sync_copy(floors_hbm.at[pl.ds(base, _BPW)], idx_v)
    cps = []
    for c in range(_NICH):
        cp = pltpu.async_copy(
            p_hbm.at[idx_v.at[pl.ds(c * _ICH, _ICH)]],
            rows_v.at[pl.ds(c * _ICH, _ICH)], sem)
        cps.append(cp)
    for cp in cps:
        cp.wait()
    pltpu.sync_copy(rows_v, out_hbm.at[pl.ds(base, _BPW)])


def _sc_gather(p, floors):
    mesh = plsc.VectorSubcoreMesh(core_axis_name="c", subcore_axis_name="s")
    k = functools.partial(
        pl.kernel, mesh=mesh,
        out_type=jax.ShapeDtypeStruct((B, INPUT_DIM), jnp.float32),
        scratch_types=[
            pltpu.VMEM((_BPW,), jnp.int32),
            pltpu.VMEM((_BPW, INPUT_DIM), jnp.float32),
            pltpu.SemaphoreType.DMA,
        ],
    )(_sc_gather_kernel)
    return k(p, floors)


def _tc_kernel(enc_ref, g_ref, w1_ref, out_ref):
    dense = jax.lax.dot_general(
        enc_ref[...], w1_ref[...],
        dimension_numbers=(((1,), (1,)), ((), ())),
        preferred_element_type=jnp.float32,
    )
    out_ref[...] = dense + g_ref[...]


def kernel(encodings, src_floors, emb_table, W, b):
    w1 = W[:, :INPUT_DIM]
    w2 = W[:, INPUT_DIM:]
    b2 = b.reshape(1, INPUT_DIM)
    p = pl.pallas_call(
        _p_kernel,
        out_shape=jax.ShapeDtypeStruct((2, INPUT_DIM), jnp.float32),
    )(emb_table, w2, b2)
    gathered = _sc_gather(p, src_floors.astype(jnp.int32))
    return pl.pallas_call(
        _tc_kernel,
        grid=(GRID,),
        in_specs=[
            pl.BlockSpec((BLK, INPUT_DIM), lambda i: (i, 0)),
            pl.BlockSpec((BLK, INPUT_DIM), lambda i: (i, 0)),
            pl.BlockSpec((INPUT_DIM, INPUT_DIM), lambda i: (0, 0)),
        ],
        out_specs=pl.BlockSpec((BLK, INPUT_DIM), lambda i: (i, 0)),
        out_shape=jax.ShapeDtypeStruct((B, INPUT_DIM), jnp.float32),
        compiler_params=pltpu.CompilerParams(
            dimension_semantics=("arbitrary",),
        ),
    )(encodings, gathered, w1)


# speedup vs baseline: 34.5567x; 34.5567x over previous
import jax
import jax.numpy as jnp
from jax.experimental import pallas as pl
from jax.experimental.pallas import tpu as pltpu

B = 16384
INPUT_DIM = 128
EMBED_DIM = 16
LCH = 2048
NL = B // LCH
CCH = 4096
NCC = B // CCH
LPC = CCH // LCH


def _fused_kernel(enc_hbm, floors_ref, emb_ref, w1_ref, w2_ref, b_ref,
                  out_hbm, enc_buf, out_buf, lsem, ssem):
    loads = []
    for c in range(NL):
        cp = pltpu.make_async_copy(
            enc_hbm.at[pl.ds(c * LCH, LCH), :],
            enc_buf.at[c // LPC, pl.ds((c % LPC) * LCH, LCH), :],
            lsem.at[c])
        cp.start()
        loads.append(cp)

    p = jax.lax.dot_general(
        emb_ref[...], w2_ref[...],
        dimension_numbers=(((1,), (1,)), ((), ())),
        preferred_element_type=jnp.float32,
    )
    base = p[0:1, :] + b_ref[...]
    pdiff = p[1:2, :] - p[0:1, :]

    stores = []
    for k in range(NCC):
        for j in range(LPC):
            loads[k * LPC + j].wait()
        dense = jax.lax.dot_general(
            enc_buf[k], w1_ref[...],
            dimension_numbers=(((1,), (1,)), ((), ())),
            preferred_element_type=jnp.float32,
        )
        f = floors_ref[k, 0, :].astype(jnp.float32)[:, None]
        out_buf[k] = (dense + base) + f * pdiff
        st = pltpu.make_async_copy(
            out_buf.at[k], out_hbm.at[pl.ds(k * CCH, CCH), :], ssem.at[k])
        st.start()
        stores.append(st)

    for st in stores:
        st.wait()


def kernel(encodings, src_floors, emb_table, W, b):
    w1 = W[:, :INPUT_DIM]
    w2 = W[:, INPUT_DIM:]
    floors3 = src_floors.astype(jnp.int32).reshape(NCC, 1, CCH)
    b2 = b.reshape(1, INPUT_DIM)
    return pl.pallas_call(
        _fused_kernel,
        in_specs=[
            pl.BlockSpec(memory_space=pl.ANY),
            pl.BlockSpec(memory_space=pltpu.MemorySpace.VMEM),
            pl.BlockSpec(memory_space=pltpu.MemorySpace.VMEM),
            pl.BlockSpec(memory_space=pltpu.MemorySpace.VMEM),
            pl.BlockSpec(memory_space=pltpu.MemorySpace.VMEM),
            pl.BlockSpec(memory_space=pltpu.MemorySpace.VMEM),
        ],
        out_specs=pl.BlockSpec(memory_space=pl.ANY),
        out_shape=jax.ShapeDtypeStruct((B, INPUT_DIM), jnp.float32),
        scratch_shapes=[
            pltpu.VMEM((NCC, CCH, INPUT_DIM), jnp.float32),
            pltpu.VMEM((NCC, CCH, INPUT_DIM), jnp.float32),
            pltpu.SemaphoreType.DMA((NL,)),
            pltpu.SemaphoreType.DMA((NCC,)),
        ],
    )(encodings, floors3, emb_table, w1, w2, b2)
